# baseline (speedup 1.0000x reference)
import functools

import jax
import jax.numpy as jnp
from jax import lax
from jax.experimental import pallas as pl
from jax.experimental.pallas import tpu as pltpu
from jax.experimental.pallas import tpu_sc as plsc

D = 128
LANES = 16
CHUNK = 64
NT = 32
SUB_ROWS = 640


def _mm1_body(x_ref, w_ref, asrc_ref, adst_ref, h_ref, scal_ref):
    h = jnp.dot(x_ref[...], w_ref[...], preferred_element_type=jnp.float32)
    h_ref[...] = h
    scal_ref[0, :] = jnp.sum(h * asrc_ref[...], axis=1)
    scal_ref[1, :] = jnp.sum(h * adst_ref[...], axis=1)


def _combine_mm_body(acc_ref, den_ref, b_ref, w_ref, asrc_ref, adst_ref,
                     h_ref, scal_ref):
    den = jnp.sum(den_ref[...], axis=0) + 1e-16
    x = (acc_ref[0] + acc_ref[1]) / den[:, None] + b_ref[...]
    x = jnp.maximum(x, 0.0)
    h = jnp.dot(x, w_ref[...], preferred_element_type=jnp.float32)
    h_ref[...] = h
    scal_ref[0, :] = jnp.sum(h * asrc_ref[...], axis=1)
    scal_ref[1, :] = jnp.sum(h * adst_ref[...], axis=1)


def _final_body(acc_ref, den_ref, b_ref, ab_ref, ag_ref,
                bn2g_ref, bn2b_ref, bn2m_ref, bn2v_ref,
                agg_ref, agb_ref, agm_ref, agv_ref,
                fcw_ref, fcb_ref, agfcw_ref, agfcb_ref,
                oab_ref, oag_ref):
    nab = ab_ref.shape[0]
    nag = ag_ref.shape[0]
    den = jnp.sum(den_ref[...], axis=0) + 1e-16
    x2 = (acc_ref[0] + acc_ref[1]) / den[:, None] + b_ref[...]
    xab = jnp.concatenate([x2[:nab], ab_ref[...]], axis=1)
    xab = (xab - bn2m_ref[...]) / jnp.sqrt(bn2v_ref[...] + 1e-5) * bn2g_ref[...] + bn2b_ref[...]
    xab = jnp.maximum(xab, 0.0)
    oab_ref[...] = jnp.dot(xab, fcw_ref[...], preferred_element_type=jnp.float32) + fcb_ref[0, 0]
    xg = jnp.concatenate([x2[nab:nab + nag], ag_ref[...]], axis=1)
    xg = (xg - agm_ref[...]) / jnp.sqrt(agv_ref[...] + 1e-5) * agg_ref[...] + agb_ref[...]
    xg = jnp.maximum(xg, 0.0)
    oag_ref[...] = jnp.dot(xg, agfcw_ref[...], preferred_element_type=jnp.float32) + agfcb_ref[0, 0]


def _make_sc_scalar_kernel(n_pad, n_chunks):
    mesh = plsc.VectorSubcoreMesh(core_axis_name="c", subcore_axis_name="s")
    ept = n_chunks * CHUNK

    @functools.partial(
        pl.kernel,
        mesh=mesh,
        compiler_params=pltpu.CompilerParams(needs_layout_passes=False,
                                             use_tc_tiling_on_sc=False),
        out_type=[
            jax.ShapeDtypeStruct((NT, ept), jnp.float32),
            jax.ShapeDtypeStruct((NT, n_pad), jnp.float32),
        ],
        scratch_types=[
            pltpu.VMEM((n_pad,), jnp.float32),
            pltpu.VMEM((n_pad,), jnp.float32),
            pltpu.VMEM((n_pad,), jnp.float32),
            pltpu.VMEM((ept,), jnp.int32),
            pltpu.VMEM((ept,), jnp.float32),
        ],
    )
    def sc_scalar(asrc_hbm, adst_hbm, eidx_hbm, exf_out, den_out,
                  asrc_t, adst_t, denom_t, idx_t, ex_t):
        c = lax.axis_index("c")
        s = lax.axis_index("s")
        wid = s * 2 + c

        pltpu.sync_copy(asrc_hbm, asrc_t)
        pltpu.sync_copy(adst_hbm, adst_t)
        pltpu.sync_copy(eidx_hbm.at[wid], idx_t)

        zero16 = jnp.zeros((LANES,), jnp.float32)

        def zden(i, carry):
            denom_t[pl.ds(i * LANES, LANES)] = zero16
            return carry
        lax.fori_loop(0, n_pad // LANES, zden, 0)

        def grp(g):
            w = idx_t[pl.ds(g * LANES, LANES)]
            sidx = w & 0xFFFF
            didx = lax.shift_right_logical(w, 16)
            a = plsc.load_gather(asrc_t, [sidx]) + plsc.load_gather(adst_t, [didx])
            al = jnp.where(a >= 0.0, a, a * 0.2)
            ex = jnp.exp(al)
            plsc.addupdate_scatter(denom_t, [didx], ex)
            ex_t[pl.ds(g * LANES, LANES)] = ex
        plsc.parallel_loop(0, ept // LANES, 1, unroll=4)(grp)

        pltpu.sync_copy(ex_t, exf_out.at[wid])
        pltpu.sync_copy(denom_t, den_out.at[wid])

    return sc_scalar


def _make_sc_feature_kernel(n_pad, n_chunks):
    mesh = plsc.VectorSubcoreMesh(core_axis_name="c", subcore_axis_name="s")
    ept = n_chunks * CHUNK
    assert n_chunks % 2 == 0 and n_chunks >= 4

    @functools.partial(
        pl.kernel,
        mesh=mesh,
        compiler_params=pltpu.CompilerParams(needs_layout_passes=False,
                                             use_tc_tiling_on_sc=False),
        out_type=jax.ShapeDtypeStruct((2, n_pad, D), jnp.float32),
        scratch_types=[
            pltpu.VMEM((ept,), jnp.int32),
            pltpu.VMEM((ept,), jnp.float32),
            [pltpu.VMEM((2, CHUNK), jnp.int32)] * 2,
            [pltpu.VMEM((CHUNK, D // 2), jnp.int32)] * 2,
            [pltpu.VMEM((CHUNK, D), jnp.float32)] * 2,
            pltpu.VMEM_SHARED((n_pad, D), jnp.float32),
            [pltpu.SemaphoreType.DMA] * 2,
            [pltpu.SemaphoreType.DMA] * 2,
        ],
    )
    def sc_feature(h_hbm, eidx_hbm, exf_hbm, acc_out,
                   idx_t, ex_t, idxu, rows_bf, rows, acc_sh, gsem, ssem):
        c = lax.axis_index("c")
        s = lax.axis_index("s")
        wid = s * 2 + c

        pltpu.sync_copy(eidx_hbm.at[wid], idx_t)
        pltpu.sync_copy(exf_hbm.at[wid], ex_t)

        zero16 = jnp.zeros((LANES,), jnp.float32)

        def zrow(i, carry):
            for j in range(D // LANES):
                rows[0][i, pl.ds(j * LANES, LANES)] = zero16
            return carry
        lax.fori_loop(0, CHUNK, zrow, 0)

        for t in range(SUB_ROWS // CHUNK):
            pltpu.sync_copy(rows[0],
                            acc_sh.at[pl.ds(s * SUB_ROWS + t * CHUNK, CHUNK)])
        plsc.subcore_barrier()

        def unpack(k, X):
            for g in range(CHUNK // LANES):
                w = idx_t[pl.ds(k * CHUNK + g * LANES, LANES)]
                idxu[X][0, pl.ds(g * LANES, LANES)] = w & 0xFFFF
                idxu[X][1, pl.ds(g * LANES, LANES)] = (
                    lax.shift_right_logical(w, 16))

        def gather(k, P):
            pltpu.async_copy(h_hbm.at[idxu[P].at[0]], rows_bf[P], gsem[P])

        def wait_gather(P):
            pltpu.make_async_copy(h_hbm.at[idxu[P].at[0]], rows_bf[P],
                                  gsem[P]).wait()

        def scatter(P):
            pltpu.async_copy(rows[P], acc_sh.at[idxu[P].at[1]], ssem[P],
                             add=True)

        def wait_scatter(P):
            pltpu.make_async_copy(rows[P], acc_sh.at[idxu[P].at[1]],
                                  ssem[P]).wait()

        def compute(k, P):
            rowsP = rows[P]
            rowsbP = rows_bf[P]

            def scale(e):
                exb = plsc.load_gather(
                    ex_t, [jnp.full((LANES,), k * CHUNK + e, jnp.int32)])
                for j2 in range(D // (2 * LANES)):
                    w = rowsbP[e, pl.ds(j2 * LANES, LANES)]
                    v = plsc.bitcast(w, jnp.bfloat16)
                    a, b = plsc.unpack(v, format=plsc.PackFormat.INTERLEAVED)
                    rowsP[e, pl.ds(j2 * 2 * LANES, LANES)] = a * exb
                    rowsP[e, pl.ds(j2 * 2 * LANES + LANES, LANES)] = b * exb
            plsc.parallel_loop(0, CHUNK, 1, unroll=4)(scale)

        unpack(0, 0)
        gather(0, 0)
        wait_gather(0)
        unpack(1, 1)
        gather(1, 1)
        compute(0, 0)
        scatter(0)
        wait_gather(1)
        wait_scatter(0)
        unpack(2, 0)
        gather(2, 0)
        compute(1, 1)
        scatter(1)

        def super_body(j, carry):
            for q, P in ((0, 0), (1, 1)):
                k = 2 * j + q
                Q = 1 - P
                wait_gather(P)
                wait_scatter(Q)
                unpack(k + 1, Q)
                gather(k + 1, Q)
                compute(k, P)
                scatter(P)
            return carry
        lax.fori_loop(1, n_chunks // 2 - 1, super_body, 0)

        k = n_chunks - 2
        wait_gather(0)
        wait_scatter(1)
        unpack(k + 1, 1)
        gather(k + 1, 1)
        compute(k, 0)
        scatter(0)
        wait_gather(1)
        wait_scatter(0)
        compute(k + 1, 1)
        scatter(1)
        wait_scatter(1)

        plsc.subcore_barrier()
        for t in range(SUB_ROWS // CHUNK):
            off = s * SUB_ROWS + t * CHUNK
            pltpu.sync_copy(acc_sh.at[pl.ds(off, CHUNK)],
                            acc_out.at[c, pl.ds(off, CHUNK)])

    return sc_feature


def kernel(selected_ab, x_ag, edge_index, W1, a_src1, a_dst1, b1,
           W2, a_src2, a_dst2, b2,
           bn2_g, bn2_b, bn2_m, bn2_v, ag_g, ag_b, ag_m, ag_v,
           fc_w, fc_b, agfc_w, agfc_b):
    nab = selected_ab.shape[0]
    nag = x_ag.shape[0]
    n = nab + nag
    e_tot = edge_index.shape[1] + n
    n_chunks = (-(-e_tot // (NT * CHUNK)) + 3) // 4 * 4
    ept = n_chunks * CHUNK
    pad_e = NT * ept - e_tot
    n_pad = -(-n // SUB_ROWS) * SUB_ROWS

    x = jnp.concatenate(
        [selected_ab, x_ag, jnp.zeros((n_pad - n, D), jnp.float32)], axis=0)
    loops = jnp.arange(n, dtype=jnp.int32)
    src = jnp.concatenate(
        [edge_index[0], loops, jnp.zeros((pad_e,), jnp.int32)])
    dst = jnp.concatenate(
        [edge_index[1], loops, jnp.full((pad_e,), n, jnp.int32)])
    eidx = (src | (dst << 16)).reshape(NT, n_chunks * CHUNK)

    mm1 = pl.pallas_call(
        _mm1_body,
        out_shape=[jax.ShapeDtypeStruct((n_pad, D), jnp.float32),
                   jax.ShapeDtypeStruct((2, n_pad), jnp.float32)],
    )
    h1, scal1 = mm1(x, W1, a_src1.reshape(1, D), a_dst1.reshape(1, D))

    sc_scalar = _make_sc_scalar_kernel(n_pad, n_chunks)
    sc_feature = _make_sc_feature_kernel(n_pad, n_chunks)
    def _interleave_bf16(h):
        hp = h.reshape(n_pad, D // 32, 2, 16).swapaxes(2, 3).reshape(n_pad, D)
        hb = hp.astype(jnp.bfloat16).reshape(n_pad, D // 2, 2)
        return jax.lax.bitcast_convert_type(hb, jnp.int32)

    exf1, den1 = sc_scalar(scal1[0], scal1[1], eidx)
    acc1 = sc_feature(_interleave_bf16(h1), eidx, exf1)

    mm2 = pl.pallas_call(
        _combine_mm_body,
        out_shape=[jax.ShapeDtypeStruct((n_pad, D), jnp.float32),
                   jax.ShapeDtypeStruct((2, n_pad), jnp.float32)],
    )
    h2, scal2 = mm2(acc1, den1, b1.reshape(1, D), W2,
                    a_src2.reshape(1, D), a_dst2.reshape(1, D))

    exf2, den2 = sc_scalar(scal2[0], scal2[1], eidx)
    acc2 = sc_feature(_interleave_bf16(h2), eidx, exf2)

    fin = pl.pallas_call(
        _final_body,
        out_shape=[jax.ShapeDtypeStruct((nab, 1), jnp.float32),
                   jax.ShapeDtypeStruct((nag, 1), jnp.float32)],
    )
    yab, yg = fin(acc2, den2, b2.reshape(1, D), selected_ab, x_ag,
                  bn2_g.reshape(1, 2 * D), bn2_b.reshape(1, 2 * D),
                  bn2_m.reshape(1, 2 * D), bn2_v.reshape(1, 2 * D),
                  ag_g.reshape(1, 2 * D), ag_b.reshape(1, 2 * D),
                  ag_m.reshape(1, 2 * D), ag_v.reshape(1, 2 * D),
                  fc_w, fc_b.reshape(1, 1), agfc_w, agfc_b.reshape(1, 1))
    return (yab.reshape(-1), yg.reshape(-1))


# speedup vs baseline: 1.0686x; 1.0686x over previous
import functools

import jax
import jax.numpy as jnp
from jax import lax
from jax.experimental import pallas as pl
from jax.experimental.pallas import tpu as pltpu
from jax.experimental.pallas import tpu_sc as plsc

D = 128
LANES = 16
CHUNK = 64
NT = 32
SUB_ROWS = 640


def _mm1_body(x_ref, w_ref, asrc_ref, adst_ref, h_ref, scal_ref):
    h = jnp.dot(x_ref[...], w_ref[...], preferred_element_type=jnp.float32)
    h_ref[...] = h
    scal_ref[0, :] = jnp.sum(h * asrc_ref[...], axis=1)
    scal_ref[1, :] = jnp.sum(h * adst_ref[...], axis=1)


def _combine_mm_body(acc_ref, den_ref, b_ref, w_ref, asrc_ref, adst_ref,
                     h_ref, scal_ref):
    den = jnp.sum(den_ref[...], axis=0) + 1e-16
    x = (acc_ref[0] + acc_ref[1]) / den[:, None] + b_ref[...]
    x = jnp.maximum(x, 0.0)
    h = jnp.dot(x, w_ref[...], preferred_element_type=jnp.float32)
    h_ref[...] = h
    scal_ref[0, :] = jnp.sum(h * asrc_ref[...], axis=1)
    scal_ref[1, :] = jnp.sum(h * adst_ref[...], axis=1)


def _final_body(acc_ref, den_ref, b_ref, ab_ref, ag_ref,
                bn2g_ref, bn2b_ref, bn2m_ref, bn2v_ref,
                agg_ref, agb_ref, agm_ref, agv_ref,
                fcw_ref, fcb_ref, agfcw_ref, agfcb_ref,
                oab_ref, oag_ref):
    nab = ab_ref.shape[0]
    nag = ag_ref.shape[0]
    den = jnp.sum(den_ref[...], axis=0) + 1e-16
    x2 = (acc_ref[0] + acc_ref[1]) / den[:, None] + b_ref[...]
    xab = jnp.concatenate([x2[:nab], ab_ref[...]], axis=1)
    xab = (xab - bn2m_ref[...]) / jnp.sqrt(bn2v_ref[...] + 1e-5) * bn2g_ref[...] + bn2b_ref[...]
    xab = jnp.maximum(xab, 0.0)
    oab_ref[...] = jnp.dot(xab, fcw_ref[...], preferred_element_type=jnp.float32) + fcb_ref[0, 0]
    xg = jnp.concatenate([x2[nab:nab + nag], ag_ref[...]], axis=1)
    xg = (xg - agm_ref[...]) / jnp.sqrt(agv_ref[...] + 1e-5) * agg_ref[...] + agb_ref[...]
    xg = jnp.maximum(xg, 0.0)
    oag_ref[...] = jnp.dot(xg, agfcw_ref[...], preferred_element_type=jnp.float32) + agfcb_ref[0, 0]


def _make_sc_scalar_kernel(n_pad, n_chunks):
    mesh = plsc.VectorSubcoreMesh(core_axis_name="c", subcore_axis_name="s")
    ept = n_chunks * CHUNK

    @functools.partial(
        pl.kernel,
        mesh=mesh,
        compiler_params=pltpu.CompilerParams(needs_layout_passes=False),
        out_type=[
            jax.ShapeDtypeStruct((NT, ept), jnp.float32),
            jax.ShapeDtypeStruct((NT, n_pad), jnp.float32),
        ],
        scratch_types=[
            pltpu.VMEM((n_pad,), jnp.float32),
            pltpu.VMEM((n_pad,), jnp.float32),
            pltpu.VMEM((n_pad,), jnp.float32),
            pltpu.VMEM((ept,), jnp.int32),
            pltpu.VMEM((ept,), jnp.float32),
        ],
    )
    def sc_scalar(asrc_hbm, adst_hbm, eidx_hbm, exf_out, den_out,
                  asrc_t, adst_t, denom_t, idx_t, ex_t):
        c = lax.axis_index("c")
        s = lax.axis_index("s")
        wid = s * 2 + c

        pltpu.sync_copy(asrc_hbm, asrc_t)
        pltpu.sync_copy(adst_hbm, adst_t)
        pltpu.sync_copy(eidx_hbm.at[wid], idx_t)

        zero16 = jnp.zeros((LANES,), jnp.float32)

        def zden(i, carry):
            denom_t[pl.ds(i * LANES, LANES)] = zero16
            return carry
        lax.fori_loop(0, n_pad // LANES, zden, 0)

        def grp(g):
            w = idx_t[pl.ds(g * LANES, LANES)]
            sidx = w & 0xFFFF
            didx = lax.shift_right_logical(w, 16)
            a = plsc.load_gather(asrc_t, [sidx]) + plsc.load_gather(adst_t, [didx])
            al = jnp.where(a >= 0.0, a, a * 0.2)
            ex = jnp.exp(al)
            plsc.addupdate_scatter(denom_t, [didx], ex)
            ex_t[pl.ds(g * LANES, LANES)] = ex
        plsc.parallel_loop(0, ept // LANES, 1, unroll=4)(grp)

        pltpu.sync_copy(ex_t, exf_out.at[wid])
        pltpu.sync_copy(denom_t, den_out.at[wid])

    return sc_scalar


def _make_sc_feature_kernel(n_pad, n_chunks):
    mesh = plsc.VectorSubcoreMesh(core_axis_name="c", subcore_axis_name="s")
    ept = n_chunks * CHUNK
    assert n_chunks % 2 == 0 and n_chunks >= 4

    @functools.partial(
        pl.kernel,
        mesh=mesh,
        compiler_params=pltpu.CompilerParams(needs_layout_passes=False,
                                             use_tc_tiling_on_sc=False),
        out_type=jax.ShapeDtypeStruct((2, n_pad, D), jnp.float32),
        scratch_types=[
            pltpu.VMEM((ept,), jnp.int32),
            pltpu.VMEM((ept,), jnp.float32),
            [pltpu.VMEM((2, CHUNK), jnp.int32)] * 2,
            [pltpu.VMEM((CHUNK, D // 2), jnp.int32)] * 2,
            [pltpu.VMEM((CHUNK, D), jnp.float32)] * 2,
            pltpu.VMEM_SHARED((n_pad, D), jnp.float32),
            [pltpu.SemaphoreType.DMA] * 2,
            [pltpu.SemaphoreType.DMA] * 2,
        ],
    )
    def sc_feature(h_hbm, eidx_hbm, exf_hbm, acc_out,
                   idx_t, ex_t, idxu, rows_bf, rows, acc_sh, gsem, ssem):
        c = lax.axis_index("c")
        s = lax.axis_index("s")
        wid = s * 2 + c

        pltpu.sync_copy(eidx_hbm.at[wid], idx_t)
        pltpu.sync_copy(exf_hbm.at[wid], ex_t)

        zero16 = jnp.zeros((LANES,), jnp.float32)

        def zrow(i, carry):
            for j in range(D // LANES):
                rows[0][i, pl.ds(j * LANES, LANES)] = zero16
            return carry
        lax.fori_loop(0, CHUNK, zrow, 0)

        for t in range(SUB_ROWS // CHUNK):
            pltpu.sync_copy(rows[0],
                            acc_sh.at[pl.ds(s * SUB_ROWS + t * CHUNK, CHUNK)])
        plsc.subcore_barrier()

        def unpack(k, X):
            for g in range(CHUNK // LANES):
                w = idx_t[pl.ds(k * CHUNK + g * LANES, LANES)]
                idxu[X][0, pl.ds(g * LANES, LANES)] = w & 0xFFFF
                idxu[X][1, pl.ds(g * LANES, LANES)] = (
                    lax.shift_right_logical(w, 16))

        def gather(k, P):
            pltpu.async_copy(h_hbm.at[idxu[P].at[0]], rows_bf[P], gsem[P])

        def wait_gather(P):
            pltpu.make_async_copy(h_hbm.at[idxu[P].at[0]], rows_bf[P],
                                  gsem[P]).wait()

        def scatter(P):
            pltpu.async_copy(rows[P], acc_sh.at[idxu[P].at[1]], ssem[P],
                             add=True)

        def wait_scatter(P):
            pltpu.make_async_copy(rows[P], acc_sh.at[idxu[P].at[1]],
                                  ssem[P]).wait()

        def compute(k, P):
            rowsP = rows[P]
            rowsbP = rows_bf[P]

            def scale(e):
                exb = plsc.load_gather(
                    ex_t, [jnp.full((LANES,), k * CHUNK + e, jnp.int32)])
                for j2 in range(D // (2 * LANES)):
                    w = rowsbP[e, pl.ds(j2 * LANES, LANES)]
                    v = plsc.bitcast(w, jnp.bfloat16)
                    a, b = plsc.unpack(v, format=plsc.PackFormat.INTERLEAVED)
                    rowsP[e, pl.ds(j2 * 2 * LANES, LANES)] = a * exb
                    rowsP[e, pl.ds(j2 * 2 * LANES + LANES, LANES)] = b * exb
            plsc.parallel_loop(0, CHUNK, 1, unroll=4)(scale)

        unpack(0, 0)
        gather(0, 0)
        wait_gather(0)
        unpack(1, 1)
        gather(1, 1)
        compute(0, 0)
        scatter(0)
        wait_gather(1)
        wait_scatter(0)
        unpack(2, 0)
        gather(2, 0)
        compute(1, 1)
        scatter(1)

        def super_body(j, carry):
            for q, P in ((0, 0), (1, 1)):
                k = 2 * j + q
                Q = 1 - P
                wait_gather(P)
                wait_scatter(Q)
                unpack(k + 1, Q)
                gather(k + 1, Q)
                compute(k, P)
                scatter(P)
            return carry
        lax.fori_loop(1, n_chunks // 2 - 1, super_body, 0)

        k = n_chunks - 2
        wait_gather(0)
        wait_scatter(1)
        unpack(k + 1, 1)
        gather(k + 1, 1)
        compute(k, 0)
        scatter(0)
        wait_gather(1)
        wait_scatter(0)
        compute(k + 1, 1)
        scatter(1)
        wait_scatter(1)

        plsc.subcore_barrier()
        for t in range(SUB_ROWS // CHUNK):
            off = s * SUB_ROWS + t * CHUNK
            pltpu.sync_copy(acc_sh.at[pl.ds(off, CHUNK)],
                            acc_out.at[c, pl.ds(off, CHUNK)])

    return sc_feature


def kernel(selected_ab, x_ag, edge_index, W1, a_src1, a_dst1, b1,
           W2, a_src2, a_dst2, b2,
           bn2_g, bn2_b, bn2_m, bn2_v, ag_g, ag_b, ag_m, ag_v,
           fc_w, fc_b, agfc_w, agfc_b):
    nab = selected_ab.shape[0]
    nag = x_ag.shape[0]
    n = nab + nag
    e_tot = edge_index.shape[1] + n
    n_chunks = (-(-e_tot // (NT * CHUNK)) + 3) // 4 * 4
    ept = n_chunks * CHUNK
    pad_e = NT * ept - e_tot
    n_pad = -(-n // SUB_ROWS) * SUB_ROWS

    x = jnp.concatenate(
        [selected_ab, x_ag, jnp.zeros((n_pad - n, D), jnp.float32)], axis=0)
    loops = jnp.arange(n, dtype=jnp.int32)
    src = jnp.concatenate(
        [edge_index[0], loops, jnp.zeros((pad_e,), jnp.int32)])
    dst = jnp.concatenate(
        [edge_index[1], loops, jnp.full((pad_e,), n, jnp.int32)])
    eidx = (src | (dst << 16)).reshape(NT, n_chunks * CHUNK)

    mm1 = pl.pallas_call(
        _mm1_body,
        out_shape=[jax.ShapeDtypeStruct((n_pad, D), jnp.float32),
                   jax.ShapeDtypeStruct((2, n_pad), jnp.float32)],
    )
    h1, scal1 = mm1(x, W1, a_src1.reshape(1, D), a_dst1.reshape(1, D))

    sc_scalar = _make_sc_scalar_kernel(n_pad, n_chunks)
    sc_feature = _make_sc_feature_kernel(n_pad, n_chunks)
    def _interleave_bf16(h):
        hp = h.reshape(n_pad, D // 32, 2, 16).swapaxes(2, 3).reshape(n_pad, D)
        hb = hp.astype(jnp.bfloat16).reshape(n_pad, D // 2, 2)
        return jax.lax.bitcast_convert_type(hb, jnp.int32)

    exf1, den1 = sc_scalar(scal1[0], scal1[1], eidx)
    acc1 = sc_feature(_interleave_bf16(h1), eidx, exf1)

    mm2 = pl.pallas_call(
        _combine_mm_body,
        out_shape=[jax.ShapeDtypeStruct((n_pad, D), jnp.float32),
                   jax.ShapeDtypeStruct((2, n_pad), jnp.float32)],
    )
    h2, scal2 = mm2(acc1, den1, b1.reshape(1, D), W2,
                    a_src2.reshape(1, D), a_dst2.reshape(1, D))

    exf2, den2 = sc_scalar(scal2[0], scal2[1], eidx)
    acc2 = sc_feature(_interleave_bf16(h2), eidx, exf2)

    fin = pl.pallas_call(
        _final_body,
        out_shape=[jax.ShapeDtypeStruct((nab, 1), jnp.float32),
                   jax.ShapeDtypeStruct((nag, 1), jnp.float32)],
    )
    yab, yg = fin(acc2, den2, b2.reshape(1, D), selected_ab, x_ag,
                  bn2_g.reshape(1, 2 * D), bn2_b.reshape(1, 2 * D),
                  bn2_m.reshape(1, 2 * D), bn2_v.reshape(1, 2 * D),
                  ag_g.reshape(1, 2 * D), ag_b.reshape(1, 2 * D),
                  ag_m.reshape(1, 2 * D), ag_v.reshape(1, 2 * D),
                  fc_w, fc_b.reshape(1, 1), agfc_w, agfc_b.reshape(1, 1))
    return (yab.reshape(-1), yg.reshape(-1))
